# Initial kernel scaffold; baseline (speedup 1.0000x reference)
import jax
import jax.numpy as jnp
from jax.experimental import pallas as pl


def kernel(S, RP, A, AP, SM, rna_table, atom_table, atom_pos_table, mod_table):
    raise NotImplementedError("write your pallas kernel here")


# speedup vs baseline: 11.0517x; 11.0517x over previous
import functools

import jax
import jax.numpy as jnp
import numpy as np
from jax.experimental import pallas as pl

N = 16384
C = 16
RNA_EMBED = 128
ATOM_EMBED = 64
NUM_RNA_TYPE = 8
NUM_ATOM_TYPE = 64
NUM_ATOM_POS = 64
EPS = 1e-10
BLK = 2048
LOG1E4 = float(np.log(10000.0))


def _body(s_ref, rp_ref, sm_ref, a_ref, ap_ref,
          rna_t_ref, atom_t_ref, atom_pos_t_ref, mod_t_ref, out_ref):
    b = BLK
    s = s_ref[0, 0, :]
    sm = sm_ref[0, 0, :]
    pos = rp_ref[0, 0, :].astype(jnp.float32)

    iota8 = jax.lax.broadcasted_iota(jnp.int32, (b, NUM_RNA_TYPE), 1)
    oh_s = (s[:, None] == iota8).astype(jnp.float32)
    rna = jnp.dot(oh_s, rna_t_ref[...], preferred_element_type=jnp.float32)
    iota3 = jax.lax.broadcasted_iota(jnp.int32, (b, 3), 1)
    oh_m = (sm[:, None] == iota3).astype(jnp.float32)
    rna = rna + jnp.dot(oh_m, mod_t_ref[...], preferred_element_type=jnp.float32)

    d2 = jax.lax.broadcasted_iota(jnp.int32, (b, RNA_EMBED), 1)
    pair = (d2 // 2).astype(jnp.float32)
    freq = jnp.exp(pair * (-2.0 * LOG1E4 / RNA_EMBED))
    ang = pos[:, None] * freq
    rna = rna + jnp.where(d2 % 2 == 0, jnp.sin(ang), jnp.cos(ang))

    iota64 = jax.lax.broadcasted_iota(jnp.int32, (b, NUM_ATOM_TYPE), 1)
    hist_a = jnp.zeros((b, NUM_ATOM_TYPE), jnp.float32)
    hist_ap = jnp.zeros((b, NUM_ATOM_TYPE), jnp.float32)
    denom = jnp.zeros((b, 1), jnp.float32)
    for c in range(C):
        a_c = a_ref[:, c][:, None]
        ap_c = ap_ref[:, c][:, None]
        m_c = (ap_c != 0).astype(jnp.float32)
        hist_a = hist_a + jnp.where(a_c == iota64, m_c, 0.0)
        hist_ap = hist_ap + jnp.where(ap_c == iota64, m_c, 0.0)
        denom = denom + m_c
    atom = jnp.dot(hist_a, atom_t_ref[...], preferred_element_type=jnp.float32)
    atom = atom + jnp.dot(hist_ap, atom_pos_t_ref[...],
                          preferred_element_type=jnp.float32)
    atom = atom * (1.0 / (denom + EPS))

    out_ref[:, 0:RNA_EMBED] = rna
    out_ref[:, RNA_EMBED:RNA_EMBED + ATOM_EMBED] = atom


@jax.jit
def _run(S, RP, A, AP, SM, rna_table, atom_table, atom_pos_table, mod_table):
    nb = N // BLK
    s3 = S.reshape(nb, 1, BLK).astype(jnp.int32)
    rp3 = RP.reshape(nb, 1, BLK).astype(jnp.int32)
    sm3 = SM.reshape(nb, 1, BLK).astype(jnp.int32)
    vec_spec = pl.BlockSpec((1, 1, BLK), lambda i: (i, 0, 0))
    mat_spec = pl.BlockSpec((BLK, C), lambda i: (i, 0))

    def table_spec(shape):
        return pl.BlockSpec(shape, lambda i: (0, 0))

    return pl.pallas_call(
        _body,
        grid=(nb,),
        in_specs=[
            vec_spec, vec_spec, vec_spec, mat_spec, mat_spec,
            table_spec((NUM_RNA_TYPE, RNA_EMBED)),
            table_spec((NUM_ATOM_TYPE, ATOM_EMBED)),
            table_spec((NUM_ATOM_POS, ATOM_EMBED)),
            table_spec((3, RNA_EMBED)),
        ],
        out_specs=pl.BlockSpec((BLK, RNA_EMBED + ATOM_EMBED), lambda i: (i, 0)),
        out_shape=jax.ShapeDtypeStruct((N, RNA_EMBED + ATOM_EMBED), jnp.float32),
    )(s3, rp3, sm3, A.astype(jnp.int32), AP.astype(jnp.int32),
      rna_table, atom_table, atom_pos_table, mod_table)


def kernel(S, RP, A, AP, SM, rna_table, atom_table, atom_pos_table, mod_table):
    return _run(S, RP, A, AP, SM, rna_table, atom_table,
                atom_pos_table, mod_table)


# speedup vs baseline: 19.9634x; 1.8064x over previous
import functools

import jax
import jax.numpy as jnp
import numpy as np
from jax import lax
from jax.experimental import pallas as pl
from jax.experimental.pallas import tpu as pltpu
from jax.experimental.pallas import tpu_sc as plsc

N = 16384
C = 16
RNA_EMBED = 128
ATOM_EMBED = 64
NUM_RNA_TYPE = 8
NUM_ATOM_TYPE = 64
NUM_ATOM_POS = 64
HIST = NUM_ATOM_TYPE + NUM_ATOM_POS
EPS = 1e-10
BLK = 2048
LOG1E4 = float(np.log(10000.0))

NUM_SC = 2
NUM_SUBCORES = 16
NW = NUM_SC * NUM_SUBCORES
CHUNK = N // NW


def _hist_body(a_hbm, ap_hbm, hist_hbm, a_v, ap_v, hist_v):
    wid = lax.axis_index("s") * NUM_SC + lax.axis_index("c")
    base = wid * CHUNK
    pltpu.sync_copy(a_hbm.at[pl.ds(base * C, CHUNK * C)], a_v)
    pltpu.sync_copy(ap_hbm.at[pl.ds(base * C, CHUNK * C)], ap_v)

    def row(n, _):
        a_vec = a_v[pl.ds(n * C, 16)]
        ap_vec = ap_v[pl.ds(n * C, 16)]
        m = jnp.where(ap_vec != 0, 1.0, 0.0).astype(jnp.float32)
        for k in range(HIST // 16):
            hist_v[pl.ds(n * HIST + 16 * k, 16)] = jnp.zeros((16,), jnp.float32)
        flat_base = jnp.full((16,), n * HIST, jnp.int32)
        plsc.addupdate_scatter(hist_v, [flat_base + a_vec], m)
        plsc.addupdate_scatter(
            hist_v, [flat_base + (ap_vec + NUM_ATOM_TYPE)], m)
        return 0

    lax.fori_loop(0, CHUNK, row, 0)
    pltpu.sync_copy(hist_v, hist_hbm.at[pl.ds(base * HIST, CHUNK * HIST)])


_hist_kernel = functools.partial(
    pl.kernel,
    mesh=plsc.VectorSubcoreMesh(core_axis_name="c", subcore_axis_name="s"),
    out_type=jax.ShapeDtypeStruct((N * HIST,), jnp.float32),
    compiler_params=pltpu.CompilerParams(needs_layout_passes=False),
    scratch_types=[
        pltpu.VMEM((CHUNK * C,), jnp.int32),
        pltpu.VMEM((CHUNK * C,), jnp.int32),
        pltpu.VMEM((CHUNK * HIST,), jnp.float32),
    ],
)(_hist_body)


def _tc_body(s_ref, rp_ref, sm_ref, hist_ref,
             rna_t_ref, mod_t_ref, combo_ref, out_ref):
    b = BLK
    s = s_ref[0, 0, :]
    sm = sm_ref[0, 0, :]
    pos = rp_ref[0, 0, :].astype(jnp.float32)

    iota8 = lax.broadcasted_iota(jnp.int32, (b, NUM_RNA_TYPE), 1)
    oh_s = (s[:, None] == iota8).astype(jnp.float32)
    rna = jnp.dot(oh_s, rna_t_ref[...], preferred_element_type=jnp.float32)
    iota3 = lax.broadcasted_iota(jnp.int32, (b, 3), 1)
    oh_m = (sm[:, None] == iota3).astype(jnp.float32)
    rna = rna + jnp.dot(oh_m, mod_t_ref[...], preferred_element_type=jnp.float32)

    d2 = lax.broadcasted_iota(jnp.int32, (b, RNA_EMBED), 1)
    pair = (d2 // 2).astype(jnp.float32)
    freq = jnp.exp(pair * (-2.0 * LOG1E4 / RNA_EMBED))
    ang = pos[:, None] * freq
    rna = rna + jnp.where(d2 % 2 == 0, jnp.sin(ang), jnp.cos(ang))

    pooled = jnp.dot(hist_ref[...], combo_ref[...],
                     preferred_element_type=jnp.float32)
    denom = pooled[:, ATOM_EMBED:ATOM_EMBED + 1]
    atom = pooled[:, 0:ATOM_EMBED] * (1.0 / (denom + EPS))

    out_ref[:, 0:RNA_EMBED] = rna
    out_ref[:, RNA_EMBED:RNA_EMBED + ATOM_EMBED] = atom


@jax.jit
def _run(S, RP, A, AP, SM, rna_table, atom_table, atom_pos_table, mod_table):
    hist = _hist_kernel(A.astype(jnp.int32).reshape(N * C),
                        AP.astype(jnp.int32).reshape(N * C)).reshape(N, HIST)

    combo = jnp.zeros((HIST, 128), jnp.float32)
    combo = combo.at[0:NUM_ATOM_TYPE, 0:ATOM_EMBED].set(atom_table)
    combo = combo.at[NUM_ATOM_TYPE:HIST, 0:ATOM_EMBED].set(atom_pos_table)
    combo = combo.at[0:NUM_ATOM_TYPE, ATOM_EMBED].set(1.0)

    nb = N // BLK
    s3 = S.reshape(nb, 1, BLK).astype(jnp.int32)
    rp3 = RP.reshape(nb, 1, BLK).astype(jnp.int32)
    sm3 = SM.reshape(nb, 1, BLK).astype(jnp.int32)
    vec_spec = pl.BlockSpec((1, 1, BLK), lambda i: (i, 0, 0))

    def table_spec(shape):
        return pl.BlockSpec(shape, lambda i: (0, 0))

    return pl.pallas_call(
        _tc_body,
        grid=(nb,),
        in_specs=[
            vec_spec, vec_spec, vec_spec,
            pl.BlockSpec((BLK, HIST), lambda i: (i, 0)),
            table_spec((NUM_RNA_TYPE, RNA_EMBED)),
            table_spec((3, RNA_EMBED)),
            table_spec((HIST, 128)),
        ],
        out_specs=pl.BlockSpec((BLK, RNA_EMBED + ATOM_EMBED), lambda i: (i, 0)),
        out_shape=jax.ShapeDtypeStruct((N, RNA_EMBED + ATOM_EMBED), jnp.float32),
    )(s3, rp3, sm3, hist, rna_table, mod_table, combo)


def kernel(S, RP, A, AP, SM, rna_table, atom_table, atom_pos_table, mod_table):
    return _run(S, RP, A, AP, SM, rna_table, atom_table,
                atom_pos_table, mod_table)


# speedup vs baseline: 20.3538x; 1.0196x over previous
import functools

import jax
import jax.numpy as jnp
import numpy as np
from jax import lax
from jax.experimental import pallas as pl
from jax.experimental.pallas import tpu as pltpu
from jax.experimental.pallas import tpu_sc as plsc

N = 16384
C = 16
RNA_EMBED = 128
ATOM_EMBED = 64
NUM_RNA_TYPE = 8
NUM_ATOM_TYPE = 64
NUM_ATOM_POS = 64
HIST = NUM_ATOM_TYPE + NUM_ATOM_POS
EPS = 1e-10
BLK = 2048
LOG1E4 = float(np.log(10000.0))

NUM_SC = 2
NUM_SUBCORES = 16
NW = NUM_SC * NUM_SUBCORES
CHUNK = N // NW


def _hist_body(a_hbm, ap_hbm, hist_hbm, a_v, ap_v, hist_v):
    wid = lax.axis_index("s") * NUM_SC + lax.axis_index("c")
    base = wid * CHUNK
    pltpu.sync_copy(a_hbm.at[pl.ds(base * C, CHUNK * C)], a_v)
    pltpu.sync_copy(ap_hbm.at[pl.ds(base * C, CHUNK * C)], ap_v)

    def row(n, _):
        a_vec = a_v[pl.ds(n * C, 16)]
        ap_vec = ap_v[pl.ds(n * C, 16)]
        m = jnp.where(ap_vec != 0, 1.0, 0.0).astype(jnp.float32)
        for k in range(HIST // 16):
            hist_v[pl.ds(n * HIST + 16 * k, 16)] = jnp.zeros((16,), jnp.float32)
        flat_base = jnp.full((16,), n * HIST, jnp.int32)
        plsc.addupdate_scatter(hist_v, [flat_base + a_vec], m)
        plsc.addupdate_scatter(
            hist_v, [flat_base + (ap_vec + NUM_ATOM_TYPE)], m)
        return 0

    lax.fori_loop(0, CHUNK, row, 0)
    pltpu.sync_copy(hist_v, hist_hbm.at[pl.ds(base * HIST, CHUNK * HIST)])


_hist_kernel = functools.partial(
    pl.kernel,
    mesh=plsc.VectorSubcoreMesh(core_axis_name="c", subcore_axis_name="s"),
    out_type=jax.ShapeDtypeStruct((N * HIST,), jnp.float32),
    compiler_params=pltpu.CompilerParams(needs_layout_passes=False),
    scratch_types=[
        pltpu.VMEM((CHUNK * C,), jnp.int32),
        pltpu.VMEM((CHUNK * C,), jnp.int32),
        pltpu.VMEM((CHUNK * HIST,), jnp.float32),
    ],
)(_hist_body)


def _tc_body(s_ref, rp_ref, sm_ref, hist_ref,
             rna_t_ref, mod_t_ref, combo_ref, fp_ref, out_ref):
    b = BLK
    s = s_ref[0, 0, :]
    sm = sm_ref[0, 0, :]
    pos = rp_ref[0, 0, :].astype(jnp.float32)

    iota8 = lax.broadcasted_iota(jnp.int32, (b, NUM_RNA_TYPE), 1)
    oh_s = (s[:, None] == iota8).astype(jnp.float32)
    rna = jnp.dot(oh_s, rna_t_ref[...], preferred_element_type=jnp.float32)
    iota3 = lax.broadcasted_iota(jnp.int32, (b, 3), 1)
    oh_m = (sm[:, None] == iota3).astype(jnp.float32)
    rna = rna + jnp.dot(oh_m, mod_t_ref[...], preferred_element_type=jnp.float32)

    ang = pos[:, None] * fp_ref[0:1, :] + fp_ref[1:2, :]
    rna = rna + jnp.sin(ang)

    pooled = jnp.dot(hist_ref[...], combo_ref[...],
                     preferred_element_type=jnp.float32)
    denom = pooled[:, ATOM_EMBED:ATOM_EMBED + 1]
    atom = pooled[:, 0:ATOM_EMBED] * (1.0 / (denom + EPS))

    out_ref[:, 0:RNA_EMBED] = rna
    out_ref[:, RNA_EMBED:RNA_EMBED + ATOM_EMBED] = atom


@jax.jit
def _run(S, RP, A, AP, SM, rna_table, atom_table, atom_pos_table, mod_table):
    hist = _hist_kernel(A.astype(jnp.int32).reshape(N * C),
                        AP.astype(jnp.int32).reshape(N * C)).reshape(N, HIST)

    combo = jnp.zeros((HIST, 128), jnp.float32)
    combo = combo.at[0:NUM_ATOM_TYPE, 0:ATOM_EMBED].set(atom_table)
    combo = combo.at[NUM_ATOM_TYPE:HIST, 0:ATOM_EMBED].set(atom_pos_table)
    combo = combo.at[0:NUM_ATOM_TYPE, ATOM_EMBED].set(1.0)

    d = np.arange(RNA_EMBED)
    freq_np = np.power(10000.0, -2.0 * (d // 2) / RNA_EMBED)
    phase_np = np.where(d % 2 == 0, 0.0, np.pi / 2)
    fp = jnp.asarray(np.stack([freq_np, phase_np]), jnp.float32)

    nb = N // BLK
    s3 = S.reshape(nb, 1, BLK).astype(jnp.int32)
    rp3 = RP.reshape(nb, 1, BLK).astype(jnp.int32)
    sm3 = SM.reshape(nb, 1, BLK).astype(jnp.int32)
    vec_spec = pl.BlockSpec((1, 1, BLK), lambda i: (i, 0, 0))

    def table_spec(shape):
        return pl.BlockSpec(shape, lambda i: (0, 0))

    return pl.pallas_call(
        _tc_body,
        grid=(nb,),
        in_specs=[
            vec_spec, vec_spec, vec_spec,
            pl.BlockSpec((BLK, HIST), lambda i: (i, 0)),
            table_spec((NUM_RNA_TYPE, RNA_EMBED)),
            table_spec((3, RNA_EMBED)),
            table_spec((HIST, 128)),
            table_spec((2, RNA_EMBED)),
        ],
        out_specs=pl.BlockSpec((BLK, RNA_EMBED + ATOM_EMBED), lambda i: (i, 0)),
        out_shape=jax.ShapeDtypeStruct((N, RNA_EMBED + ATOM_EMBED), jnp.float32),
    )(s3, rp3, sm3, hist, rna_table, mod_table, combo, fp)


def kernel(S, RP, A, AP, SM, rna_table, atom_table, atom_pos_table, mod_table):
    return _run(S, RP, A, AP, SM, rna_table, atom_table,
                atom_pos_table, mod_table)
